# baseline (device time: 31566 ns/iter reference)
import jax
import jax.numpy as jnp
from jax import lax
from jax.experimental import pallas as pl
from jax.experimental.pallas import tpu as pltpu

N_DEV = 4


def kernel(A, B):
    m, k = A.shape
    k2, n = B.shape
    m_out = m // N_DEV
    nh = n // 2


    def body(a_ref, b_ref, out_ref,
             relay_snd, relay_rcv, comb_snd, comb_rcv,
             send_sems, recv_sems):
        my = lax.axis_index("i")
        left = lax.rem(my + (N_DEV - 1), N_DEV)
        right = lax.rem(my + 1, N_DEV)
        diag_blk = lax.rem(my + 2, N_DEV)

        barrier_sem = pltpu.get_barrier_semaphore()
        for nbr in [left, right]:
            pl.semaphore_signal(
                barrier_sem, inc=1,
                device_id=(nbr,), device_id_type=pl.DeviceIdType.MESH,
            )
        pl.semaphore_wait(barrier_sem, 2)

        def a_block(blk):
            return a_ref[pl.ds(blk * m_out, m_out), :]

        b_bf = b_ref[:, :].astype(jnp.bfloat16)

        a_diag = a_block(diag_blk)
        a_diag_bf = a_diag.astype(jnp.bfloat16)
        relay_snd[1, :, :] = jnp.dot(
            a_diag_bf, b_bf[:, nh:],
            preferred_element_type=jnp.float32).astype(jnp.bfloat16)
        rel_r = pltpu.make_async_remote_copy(
            src_ref=relay_snd.at[1], dst_ref=relay_rcv.at[0],
            send_sem=send_sems.at[0], recv_sem=recv_sems.at[0],
            device_id=(right,), device_id_type=pl.DeviceIdType.MESH,
        )
        rel_r.start()
        relay_snd[0, :, :] = jnp.dot(
            a_diag_bf, b_bf[:, :nh],
            preferred_element_type=jnp.float32).astype(jnp.bfloat16)
        rel_l = pltpu.make_async_remote_copy(
            src_ref=relay_snd.at[0], dst_ref=relay_rcv.at[1],
            send_sem=send_sems.at[1], recv_sem=recv_sems.at[1],
            device_id=(left,), device_id_type=pl.DeviceIdType.MESH,
        )
        rel_l.start()

        def comb_copy(slot, dest):
            return pltpu.make_async_remote_copy(
                src_ref=comb_snd.at[slot], dst_ref=comb_rcv.at[slot],
                send_sem=send_sems.at[2 + slot], recv_sem=recv_sems.at[2 + slot],
                device_id=(dest,), device_id_type=pl.DeviceIdType.MESH,
            )

        c_r = jnp.dot(a_block(right).astype(jnp.bfloat16), b_bf,
                      preferred_element_type=jnp.float32)
        comb_snd[0, :, :] = c_r[:, :nh].astype(jnp.bfloat16)
        cmb0 = comb_copy(0, right)
        cmb0.start()
        c_l = jnp.dot(a_block(left).astype(jnp.bfloat16), b_bf,
                      preferred_element_type=jnp.float32)
        comb_snd[3, :, :] = c_l[:, nh:].astype(jnp.bfloat16)
        cmb3 = comb_copy(3, left)
        cmb3.start()
        c_own = jnp.dot(a_block(my), b_ref[:, :],
                        preferred_element_type=jnp.float32)

        rel_r.wait_recv()
        comb_snd[1, :, :] = (c_r[:, nh:]
                             + relay_rcv[0, :, :].astype(jnp.float32)
                             ).astype(jnp.bfloat16)
        cmb1 = comb_copy(1, right)
        cmb1.start()
        rel_l.wait_recv()
        comb_snd[2, :, :] = (c_l[:, :nh]
                             + relay_rcv[1, :, :].astype(jnp.float32)
                             ).astype(jnp.bfloat16)
        cmb2 = comb_copy(2, left)
        cmb2.start()

        cmb0.wait_recv()
        cmb2.wait_recv()
        out_ref[:, :nh] = (c_own[:, :nh]
                           + comb_rcv[0, :, :].astype(jnp.float32)
                           + comb_rcv[2, :, :].astype(jnp.float32))
        cmb3.wait_recv()
        cmb1.wait_recv()
        out_ref[:, nh:] = (c_own[:, nh:]
                           + comb_rcv[1, :, :].astype(jnp.float32)
                           + comb_rcv[3, :, :].astype(jnp.float32))

        for r in (rel_r, rel_l, cmb0, cmb1, cmb2, cmb3):
            r.wait_send()

    return pl.pallas_call(
        body,
        out_shape=jax.ShapeDtypeStruct((m_out, n), jnp.float32),
        in_specs=[
            pl.BlockSpec(memory_space=pltpu.VMEM),
            pl.BlockSpec(memory_space=pltpu.VMEM),
        ],
        out_specs=pl.BlockSpec(memory_space=pltpu.VMEM),
        scratch_shapes=[
            pltpu.VMEM((2, m_out, nh), jnp.bfloat16),
            pltpu.VMEM((2, m_out, nh), jnp.bfloat16),
            pltpu.VMEM((4, m_out, nh), jnp.bfloat16),
            pltpu.VMEM((4, m_out, nh), jnp.bfloat16),
            pltpu.SemaphoreType.DMA((6,)),
            pltpu.SemaphoreType.DMA((6,)),
        ],
        compiler_params=pltpu.CompilerParams(collective_id=0),
    )(A, B)
